# baseline (speedup 1.0000x reference)
import functools

import jax
import jax.numpy as jnp
from jax import lax
from jax.experimental import pallas as pl
from jax.experimental.pallas import tpu as pltpu
from jax.experimental.pallas import tpu_sc as plsc

N_FIELDS = 26
VOCAB = 100000
DIM = 32
BATCH = 16384

NC, NS, L = 2, 16, 16
NW = NC * NS
STRIPE = 4096
N_STRIPES = BATCH // STRIPE


def _make_sc_gather():
    mesh = plsc.VectorSubcoreMesh(core_axis_name="c", subcore_axis_name="s")

    @functools.partial(
        pl.kernel,
        mesh=mesh,
        out_type=jax.ShapeDtypeStruct((N_FIELDS * DIM, BATCH), jnp.float32),
        compiler_params=pltpu.CompilerParams(needs_layout_passes=False),
        scratch_types=[
            pltpu.VMEM((VOCAB,), jnp.float32),
            pltpu.VMEM((2, STRIPE), jnp.int32),
            pltpu.VMEM((2, STRIPE), jnp.float32),
            pltpu.SemaphoreType.DMA,
            pltpu.SemaphoreType.DMA,
        ],
    )
    def gather_kernel(table_hbm, xt_hbm, out_hbm, plane_v, idx_v, out_v,
                      isem, osem):
        d = lax.axis_index("s") * NC + lax.axis_index("c")

        def do_field(f, carry):
            pltpu.sync_copy(table_hbm.at[f, d], plane_v)
            p = f * DIM + d
            pltpu.async_copy(xt_hbm.at[f, pl.ds(0, STRIPE)], idx_v.at[0],
                             isem)
            for s in range(N_STRIPES):
                b = s % 2
                pltpu.make_async_copy(
                    xt_hbm.at[f, pl.ds(s * STRIPE, STRIPE)], idx_v.at[b],
                    isem).wait()
                if s + 1 < N_STRIPES:
                    pltpu.async_copy(
                        xt_hbm.at[f, pl.ds((s + 1) * STRIPE, STRIPE)],
                        idx_v.at[1 - b], isem)
                if s >= 2:
                    pltpu.make_async_copy(
                        out_v.at[b],
                        out_hbm.at[p, pl.ds((s - 2) * STRIPE, STRIPE)],
                        osem).wait()
                else:
                    @pl.when(f > 0)
                    def _wait_prev_field():
                        pltpu.make_async_copy(
                            out_v.at[b], out_hbm.at[p, pl.ds(0, STRIPE)],
                            osem).wait()

                def do_vec(i, c):
                    idx16 = idx_v[b, pl.ds(i * L, L)]
                    out_v[b, pl.ds(i * L, L)] = plsc.load_gather(
                        plane_v, [idx16])
                    return c

                lax.fori_loop(0, STRIPE // L, do_vec, 0)
                pltpu.async_copy(out_v.at[b],
                                 out_hbm.at[p, pl.ds(s * STRIPE, STRIPE)],
                                 osem)
            return carry

        lax.fori_loop(0, N_FIELDS, do_field, 0)
        pltpu.make_async_copy(out_v.at[0], out_hbm.at[0, pl.ds(0, STRIPE)],
                              osem).wait()
        pltpu.make_async_copy(out_v.at[1], out_hbm.at[0, pl.ds(0, STRIPE)],
                              osem).wait()

    return gather_kernel


_sc_gather = _make_sc_gather()


def kernel(X, tables):
    table_t = tables.transpose(0, 2, 1)
    x_t = X.T
    out_t = _sc_gather(table_t, x_t)
    return out_t.T.reshape(BATCH, N_FIELDS * DIM)


# speedup vs baseline: 1.1242x; 1.1242x over previous
import functools

import jax
import jax.numpy as jnp
from jax import lax
from jax.experimental import pallas as pl
from jax.experimental.pallas import tpu as pltpu
from jax.experimental.pallas import tpu_sc as plsc

N_FIELDS = 26
VOCAB = 100000
DIM = 32
BATCH = 16384

NC, NS, L = 2, 16, 16
NW = NC * NS
STRIPE = 4096
N_STRIPES = BATCH // STRIPE
UNROLL = 16


def _make_sc_gather():
    mesh = plsc.VectorSubcoreMesh(core_axis_name="c", subcore_axis_name="s")

    @functools.partial(
        pl.kernel,
        mesh=mesh,
        out_type=jax.ShapeDtypeStruct((N_FIELDS * DIM, BATCH), jnp.float32),
        compiler_params=pltpu.CompilerParams(needs_layout_passes=False),
        scratch_types=[
            pltpu.VMEM((VOCAB,), jnp.float32),
            pltpu.VMEM((2, STRIPE), jnp.int32),
            pltpu.VMEM((2, STRIPE), jnp.float32),
            pltpu.SemaphoreType.DMA,
            pltpu.SemaphoreType.DMA,
        ],
    )
    def gather_kernel(table_hbm, xt_hbm, out_hbm, plane_v, idx_v, out_v,
                      isem, osem):
        d = lax.axis_index("s") * NC + lax.axis_index("c")

        def do_field(f, carry):
            pltpu.sync_copy(table_hbm.at[f, d], plane_v)
            p = f * DIM + d
            pltpu.async_copy(xt_hbm.at[f, pl.ds(0, STRIPE)], idx_v.at[0],
                             isem)
            for s in range(N_STRIPES):
                b = s % 2
                pltpu.make_async_copy(
                    xt_hbm.at[f, pl.ds(s * STRIPE, STRIPE)], idx_v.at[b],
                    isem).wait()
                if s + 1 < N_STRIPES:
                    pltpu.async_copy(
                        xt_hbm.at[f, pl.ds((s + 1) * STRIPE, STRIPE)],
                        idx_v.at[1 - b], isem)
                if s >= 2:
                    pltpu.make_async_copy(
                        out_v.at[b],
                        out_hbm.at[p, pl.ds((s - 2) * STRIPE, STRIPE)],
                        osem).wait()
                else:
                    @pl.when(f > 0)
                    def _wait_prev_field():
                        pltpu.make_async_copy(
                            out_v.at[b], out_hbm.at[p, pl.ds(0, STRIPE)],
                            osem).wait()

                def do_vec(i, c):
                    base = i * (L * UNROLL)
                    for u in range(UNROLL):
                        off = base + u * L
                        idx16 = idx_v[b, pl.ds(off, L)]
                        out_v[b, pl.ds(off, L)] = plsc.load_gather(
                            plane_v, [idx16])
                    return c

                lax.fori_loop(0, STRIPE // (L * UNROLL), do_vec, 0)
                pltpu.async_copy(out_v.at[b],
                                 out_hbm.at[p, pl.ds(s * STRIPE, STRIPE)],
                                 osem)
            return carry

        lax.fori_loop(0, N_FIELDS, do_field, 0)
        pltpu.make_async_copy(out_v.at[0], out_hbm.at[0, pl.ds(0, STRIPE)],
                              osem).wait()
        pltpu.make_async_copy(out_v.at[1], out_hbm.at[0, pl.ds(0, STRIPE)],
                              osem).wait()

    return gather_kernel


_sc_gather = _make_sc_gather()


def kernel(X, tables):
    table_t = tables.transpose(0, 2, 1)
    x_t = X.T
    out_t = _sc_gather(table_t, x_t)
    return out_t.T.reshape(BATCH, N_FIELDS * DIM)


# speedup vs baseline: 1.1847x; 1.0538x over previous
import functools

import jax
import jax.numpy as jnp
from jax import lax
from jax.experimental import pallas as pl
from jax.experimental.pallas import tpu as pltpu
from jax.experimental.pallas import tpu_sc as plsc

N_FIELDS = 26
VOCAB = 100000
DIM = 32
BATCH = 16384

NC, NS, L = 2, 16, 16
NW = NC * NS
STRIPE = 4096
N_STRIPES = BATCH // STRIPE
UNROLL = 16


def _make_sc_gather():
    mesh = plsc.VectorSubcoreMesh(core_axis_name="c", subcore_axis_name="s")

    @functools.partial(
        pl.kernel,
        mesh=mesh,
        out_type=jax.ShapeDtypeStruct((N_FIELDS * DIM, BATCH), jnp.float32),
        compiler_params=pltpu.CompilerParams(needs_layout_passes=False),
        scratch_types=[
            pltpu.VMEM((VOCAB,), jnp.float32),
            pltpu.VMEM((2, STRIPE), jnp.int32),
            pltpu.VMEM((2, STRIPE), jnp.float32),
            pltpu.SemaphoreType.DMA,
            pltpu.SemaphoreType.DMA,
            pltpu.SemaphoreType.DMA,
        ],
    )
    def gather_kernel(table_hbm, xt_hbm, out_hbm, plane_v, idx_v, out_v,
                      isem, osem, psem):
        d = lax.axis_index("s") * NC + lax.axis_index("c")

        def do_field(f, carry):
            p = f * DIM + d
            pltpu.async_copy(xt_hbm.at[f, pl.ds(0, STRIPE)], idx_v.at[0],
                             isem)
            pltpu.async_copy(table_hbm.at[f, d], plane_v, psem)
            pltpu.make_async_copy(table_hbm.at[f, d], plane_v, psem).wait()
            for s in range(N_STRIPES):
                b = s % 2
                pltpu.make_async_copy(
                    xt_hbm.at[f, pl.ds(s * STRIPE, STRIPE)], idx_v.at[b],
                    isem).wait()
                if s + 1 < N_STRIPES:
                    pltpu.async_copy(
                        xt_hbm.at[f, pl.ds((s + 1) * STRIPE, STRIPE)],
                        idx_v.at[1 - b], isem)
                if s >= 2:
                    pltpu.make_async_copy(
                        out_v.at[b],
                        out_hbm.at[p, pl.ds((s - 2) * STRIPE, STRIPE)],
                        osem).wait()
                else:
                    @pl.when(f > 0)
                    def _wait_prev_field():
                        pltpu.make_async_copy(
                            out_v.at[b], out_hbm.at[p, pl.ds(0, STRIPE)],
                            osem).wait()

                def do_vec(i, c):
                    base = i * (L * UNROLL)
                    for u in range(UNROLL):
                        off = base + u * L
                        idx16 = idx_v[b, pl.ds(off, L)]
                        out_v[b, pl.ds(off, L)] = plsc.load_gather(
                            plane_v, [idx16])
                    return c

                lax.fori_loop(0, STRIPE // (L * UNROLL), do_vec, 0)
                pltpu.async_copy(out_v.at[b],
                                 out_hbm.at[p, pl.ds(s * STRIPE, STRIPE)],
                                 osem)
            return carry

        lax.fori_loop(0, N_FIELDS, do_field, 0)
        pltpu.make_async_copy(out_v.at[0], out_hbm.at[0, pl.ds(0, STRIPE)],
                              osem).wait()
        pltpu.make_async_copy(out_v.at[1], out_hbm.at[0, pl.ds(0, STRIPE)],
                              osem).wait()

    return gather_kernel


_sc_gather = _make_sc_gather()


def kernel(X, tables):
    table_t = tables.transpose(0, 2, 1)
    x_t = X.T
    out_t = _sc_gather(table_t, x_t)
    return out_t.T.reshape(BATCH, N_FIELDS * DIM)
